# baseline (speedup 1.0000x reference)
import jax
import jax.numpy as jnp
from jax.experimental import pallas as pl

HZ_ = 100
ROWS_PER_BLOCK = 3000


def _body(x_ref, sl_ref, ox_ref, osl_ref):
    ox_ref[...] = x_ref[...]
    osl_ref[...] = sl_ref[...] // HZ_


def kernel(x, seq_lens):
    bs, slen, ch = x.shape
    new_len = slen // HZ_
    new_ch = ch * HZ_
    total = bs * slen * ch
    rows = total // 128
    xf = x.reshape(rows, 128)
    sl = seq_lens.reshape(1, bs)
    grid = rows // ROWS_PER_BLOCK
    yf, osl = pl.pallas_call(
        _body,
        grid=(grid,),
        in_specs=[
            pl.BlockSpec((ROWS_PER_BLOCK, 128), lambda i: (i, 0)),
            pl.BlockSpec((1, bs), lambda i: (0, 0)),
        ],
        out_specs=[
            pl.BlockSpec((ROWS_PER_BLOCK, 128), lambda i: (i, 0)),
            pl.BlockSpec((1, bs), lambda i: (0, 0)),
        ],
        out_shape=[
            jax.ShapeDtypeStruct((rows, 128), x.dtype),
            jax.ShapeDtypeStruct((1, bs), seq_lens.dtype),
        ],
    )(xf, sl)
    return yf.reshape(bs, new_len, new_ch), osl.reshape(bs)


# speedup vs baseline: 2.7560x; 2.7560x over previous
import functools

import jax
import jax.numpy as jnp
from jax import lax
from jax.experimental import pallas as pl
from jax.experimental.pallas import tpu as pltpu
from jax.experimental.pallas import tpu_sc as plsc

HZ_ = 100
BS_ = 16
SLEN_ = 30000
CH_ = 12
NSEG_ = SLEN_ // HZ_
NCH_ = CH_ * HZ_
NVREG_ = NCH_ // 16
WIN_ = 256
SROUNDS_ = 10


def _sc_body(x_hbm, sl_hbm, idxc_hbm, idxj_hbm, out_hbm, osl_hbm,
             ts_in, ts_out, idxc_v, idxj_v, sl_v):
    info = plsc.get_sparse_core_info()
    nworkers = info.num_cores * info.num_subcores
    wid = lax.axis_index("s") * info.num_cores + lax.axis_index("c")
    iota = lax.iota(jnp.int32, 16)

    pltpu.sync_copy(idxc_hbm, idxc_v)
    pltpu.sync_copy(idxj_hbm, idxj_v)

    @pl.when(wid == 0)
    def _():
        pltpu.sync_copy(sl_hbm, sl_v)
        sl_v[...] = lax.div(sl_v[...], iota * 0 + HZ_)
        pltpu.sync_copy(sl_v, osl_hbm)

    def _seg(k, carry):
        s = wid + nworkers * k

        @pl.when(s < NSEG_)
        def _():
            t0 = s * HZ_
            a0 = pl.multiple_of((t0 // 128) * 128, 128)
            off = t0 - a0
            for c in range(CH_):
                pltpu.sync_copy(x_hbm.at[c, :, pl.ds(a0, WIN_)], ts_in.at[c])
            offv = iota * 0 + off

            def _vstep(v, carry2):
                ic = idxc_v[pl.ds(16 * v, 16)]
                ij = idxj_v[pl.ds(16 * v, 16)] + offv
                for b in range(BS_):
                    ib = iota * 0 + b
                    g = plsc.load_gather(ts_in, [ic, ib, ij])
                    ts_out[b, pl.ds(16 * v, 16)] = g
                return carry2

            lax.fori_loop(0, NVREG_, _vstep, 0)
            pltpu.sync_copy(ts_out, out_hbm.at[s])

        return carry

    lax.fori_loop(0, SROUNDS_, _seg, 0)


def kernel(x, seq_lens):
    xt = jnp.transpose(x, (2, 0, 1))
    sl = seq_lens.astype(jnp.int32)
    didx = jnp.arange(NCH_, dtype=jnp.int32)
    idxc = didx % CH_
    idxj = didx // CH_
    mesh = plsc.VectorSubcoreMesh(core_axis_name="c", subcore_axis_name="s")
    sc = functools.partial(
        pl.kernel,
        out_type=[
            jax.ShapeDtypeStruct((NSEG_, BS_, NCH_), x.dtype),
            jax.ShapeDtypeStruct((BS_,), jnp.int32),
        ],
        scratch_types=[
            pltpu.VMEM((CH_, BS_, WIN_), jnp.float32),
            pltpu.VMEM((BS_, NCH_), jnp.float32),
            pltpu.VMEM((NCH_,), jnp.int32),
            pltpu.VMEM((NCH_,), jnp.int32),
            pltpu.VMEM((BS_,), jnp.int32),
        ],
        mesh=mesh,
        compiler_params=pltpu.CompilerParams(needs_layout_passes=False),
    )(_sc_body)
    out_t, osl = sc(xt, sl, idxc, idxj)
    return jnp.transpose(out_t, (1, 0, 2)), osl.astype(seq_lens.dtype)


# speedup vs baseline: 4.4328x; 1.6084x over previous
import functools

import jax
import jax.numpy as jnp
from jax import lax
from jax.experimental import pallas as pl
from jax.experimental.pallas import tpu as pltpu
from jax.experimental.pallas import tpu_sc as plsc

HZ_ = 100
BS_ = 16
SLEN_ = 30000
CH_ = 12
NSEG_ = SLEN_ // HZ_
NCH_ = CH_ * HZ_
NVREG_ = NCH_ // 16
WIN_ = 256
NW_ = 32
SROUNDS_ = 10


def _sc_body(x_hbm, sl_hbm, idxc_hbm, idxj_hbm, out_hbm, osl_hbm,
             buf_a, buf_b, ts_out, idxc_v, idxj_v, sl_v, rsem, wsem):
    info = plsc.get_sparse_core_info()
    nworkers = info.num_cores * info.num_subcores
    wid = lax.axis_index("s") * info.num_cores + lax.axis_index("c")
    iota = lax.iota(jnp.int32, 16)

    pltpu.sync_copy(idxc_hbm, idxc_v)
    pltpu.sync_copy(idxj_hbm, idxj_v)

    @pl.when(wid == 0)
    def _():
        pltpu.sync_copy(sl_hbm, sl_v)
        sl_v[...] = lax.div(sl_v[...], iota * 0 + HZ_)
        pltpu.sync_copy(sl_v, osl_hbm)

    def _a0(s):
        return pl.multiple_of(((s * HZ_) // 128) * 128, 128)

    def _fire_reads(s, buf):
        pltpu.make_async_copy(
            x_hbm.at[:, :, pl.ds(_a0(s), WIN_)], buf, rsem
        ).start()

    def _drain_reads(s, buf):
        pltpu.make_async_copy(
            x_hbm.at[:, :, pl.ds(_a0(s), WIN_)], buf, rsem
        ).wait()

    def _gather_and_write(s, buf):
        off = s * HZ_ - _a0(s)
        offv = iota * 0 + off

        def _vstep(v, carry2):
            ic = idxc_v[pl.ds(16 * v, 16)]
            ij = idxj_v[pl.ds(16 * v, 16)] + offv
            for b in range(BS_):
                ib = iota * 0 + b
                g = plsc.load_gather(buf, [ic, ib, ij])
                ts_out[b, pl.ds(16 * v, 16)] = g
            return carry2

        lax.fori_loop(0, NVREG_, _vstep, 0)
        pltpu.make_async_copy(ts_out, out_hbm.at[s], wsem).start()

    def _phase(s, buf_cur, buf_nxt):
        @pl.when(s < NSEG_)
        def _():
            _drain_reads(s, buf_cur)
            s_nxt = s + nworkers

            @pl.when(s_nxt < NSEG_)
            def _():
                _fire_reads(s_nxt, buf_nxt)

            @pl.when(s >= nworkers)
            def _():
                pltpu.make_async_copy(ts_out, out_hbm.at[s - nworkers], wsem).wait()

            _gather_and_write(s, buf_cur)

    _fire_reads(wid, buf_a)

    def _pair(k, carry):
        s0 = wid + nworkers * (2 * k)
        _phase(s0, buf_a, buf_b)
        _phase(s0 + nworkers, buf_b, buf_a)
        return carry

    lax.fori_loop(0, SROUNDS_ // 2, _pair, 0)

    s_last = wid + nworkers * ((NSEG_ - 1 - wid) // nworkers)
    pltpu.make_async_copy(ts_out, out_hbm.at[s_last], wsem).wait()


def kernel(x, seq_lens):
    xt = jnp.transpose(x, (2, 0, 1))
    sl = seq_lens.astype(jnp.int32)
    didx = jnp.arange(NCH_, dtype=jnp.int32)
    idxc = didx % CH_
    idxj = didx // CH_
    mesh = plsc.VectorSubcoreMesh(core_axis_name="c", subcore_axis_name="s")
    sc = functools.partial(
        pl.kernel,
        out_type=[
            jax.ShapeDtypeStruct((NSEG_, BS_, NCH_), x.dtype),
            jax.ShapeDtypeStruct((BS_,), jnp.int32),
        ],
        scratch_types=[
            pltpu.VMEM((CH_, BS_, WIN_), jnp.float32),
            pltpu.VMEM((CH_, BS_, WIN_), jnp.float32),
            pltpu.VMEM((BS_, NCH_), jnp.float32),
            pltpu.VMEM((NCH_,), jnp.int32),
            pltpu.VMEM((NCH_,), jnp.int32),
            pltpu.VMEM((BS_,), jnp.int32),
            pltpu.SemaphoreType.DMA,
            pltpu.SemaphoreType.DMA,
        ],
        mesh=mesh,
        compiler_params=pltpu.CompilerParams(needs_layout_passes=False),
    )(_sc_body)
    out_t, osl = sc(xt, sl, idxc, idxj)
    return jnp.transpose(out_t, (1, 0, 2)), osl.astype(seq_lens.dtype)


# speedup vs baseline: 4.7181x; 1.0644x over previous
import functools

import jax
import jax.numpy as jnp
from jax import lax
from jax.experimental import pallas as pl
from jax.experimental.pallas import tpu as pltpu
from jax.experimental.pallas import tpu_sc as plsc

HZ_ = 100
BS_ = 16
SLEN_ = 30000
CH_ = 12
NSEG_ = SLEN_ // HZ_
NCH_ = CH_ * HZ_
NVREG_ = NCH_ // 16
WIN_ = 256
SK_ = 129
BSTR_ = CH_ * SK_
NCHUNK_ = 8
SROUNDS_ = 10


def _sc_body(x_hbm, sl_hbm, pidx_hbm, out_hbm, osl_hbm,
             buf, ts_flat, ts_out, pidx_v, sl_v, rsem, wsem):
    info = plsc.get_sparse_core_info()
    nworkers = info.num_cores * info.num_subcores
    wid = lax.axis_index("s") * info.num_cores + lax.axis_index("c")
    iota = lax.iota(jnp.int32, 16)

    pltpu.sync_copy(pidx_hbm, pidx_v)

    @pl.when(wid == 0)
    def _():
        pltpu.sync_copy(sl_hbm, sl_v)
        sl_v[...] = lax.div(sl_v[...], iota * 0 + HZ_)
        pltpu.sync_copy(sl_v, osl_hbm)

    def _a0(s):
        return pl.multiple_of(((s * HZ_) // 128) * 128, 128)

    def _read_copy(s):
        return pltpu.make_async_copy(
            x_hbm.at[:, :, pl.ds(_a0(s), WIN_)], buf, rsem
        )

    _read_copy(wid).start()

    def _seg(k, carry):
        s = wid + nworkers * k

        @pl.when(s < NSEG_)
        def _():
            t0 = s * HZ_
            off = t0 - _a0(s)
            c0 = pl.multiple_of((off // 16) * 16, 16)
            off2 = off - c0
            _read_copy(s).wait()

            def _bstep(b, carry2):
                boff = b * BSTR_
                for c in range(CH_):
                    for m in range(NCHUNK_):
                        ts_flat[pl.ds(boff + c * SK_ + 16 * m, 16)] = (
                            buf[c, b, pl.ds(c0 + 16 * m, 16)]
                        )
                return carry2

            lax.fori_loop(0, BS_, _bstep, 0)

            s_nxt = s + nworkers

            @pl.when(s_nxt < NSEG_)
            def _():
                _read_copy(s_nxt).start()

            @pl.when(s >= nworkers)
            def _():
                pltpu.make_async_copy(ts_out, out_hbm.at[s - nworkers], wsem).wait()

            off2v = iota * 0 + off2

            def _vstep(v, carry2):
                base = pidx_v[pl.ds(16 * v, 16)] + off2v
                for b in range(BS_):
                    g = plsc.load_gather(ts_flat, [base + (b * BSTR_)])
                    ts_out[b, pl.ds(16 * v, 16)] = g
                return carry2

            lax.fori_loop(0, NVREG_, _vstep, 0)
            pltpu.make_async_copy(ts_out, out_hbm.at[s], wsem).start()

        return carry

    lax.fori_loop(0, SROUNDS_, _seg, 0)

    s_last = wid + nworkers * ((NSEG_ - 1 - wid) // nworkers)
    pltpu.make_async_copy(ts_out, out_hbm.at[s_last], wsem).wait()


def kernel(x, seq_lens):
    xt = jnp.transpose(x, (2, 0, 1))
    sl = seq_lens.astype(jnp.int32)
    didx = jnp.arange(NCH_, dtype=jnp.int32)
    pidx = (didx % CH_) * SK_ + didx // CH_
    mesh = plsc.VectorSubcoreMesh(core_axis_name="c", subcore_axis_name="s")
    sc = functools.partial(
        pl.kernel,
        out_type=[
            jax.ShapeDtypeStruct((NSEG_, BS_, NCH_), x.dtype),
            jax.ShapeDtypeStruct((BS_,), jnp.int32),
        ],
        scratch_types=[
            pltpu.VMEM((CH_, BS_, WIN_), jnp.float32),
            pltpu.VMEM((BS_ * BSTR_,), jnp.float32),
            pltpu.VMEM((BS_, NCH_), jnp.float32),
            pltpu.VMEM((NCH_,), jnp.int32),
            pltpu.VMEM((BS_,), jnp.int32),
            pltpu.SemaphoreType.DMA,
            pltpu.SemaphoreType.DMA,
        ],
        mesh=mesh,
        compiler_params=pltpu.CompilerParams(needs_layout_passes=False),
    )(_sc_body)
    out_t, osl = sc(xt, sl, pidx)
    return jnp.transpose(out_t, (1, 0, 2)), osl.astype(seq_lens.dtype)


# speedup vs baseline: 4.9396x; 1.0470x over previous
import functools

import jax
import jax.numpy as jnp
from jax import lax
from jax.experimental import pallas as pl
from jax.experimental.pallas import tpu as pltpu
from jax.experimental.pallas import tpu_sc as plsc

HZ_ = 100
BS_ = 16
SLEN_ = 30000
CH_ = 12
NSEG_ = SLEN_ // HZ_
NCH_ = CH_ * HZ_
NVREG_ = NCH_ // 16
WIN_ = 256
SK_ = 129
BSTR_ = CH_ * SK_
NCHUNK_ = 8
SROUNDS_ = 10


def _sc_body(x_hbm, sl_hbm, pidx_hbm, out_hbm, osl_hbm,
             buf, ts_flat, ts_out, pidx_v, sl_v, rsem, wsem):
    info = plsc.get_sparse_core_info()
    nworkers = info.num_cores * info.num_subcores
    wid = lax.axis_index("s") * info.num_cores + lax.axis_index("c")
    iota = lax.iota(jnp.int32, 16)

    pltpu.sync_copy(pidx_hbm, pidx_v)

    @pl.when(wid == 0)
    def _():
        pltpu.sync_copy(sl_hbm, sl_v)
        sl_v[...] = lax.div(sl_v[...], iota * 0 + HZ_)
        pltpu.sync_copy(sl_v, osl_hbm)

    def _a0(s):
        return pl.multiple_of(((s * HZ_) // 128) * 128, 128)

    def _read_copy(s):
        return pltpu.make_async_copy(
            x_hbm.at[:, :, pl.ds(_a0(s), WIN_)], buf, rsem
        )

    _read_copy(wid).start()

    def _seg(k, carry):
        s = wid + nworkers * k

        @pl.when(s < NSEG_)
        def _():
            t0 = s * HZ_
            off = t0 - _a0(s)
            c0 = pl.multiple_of((off // 16) * 16, 16)
            off2 = off - c0
            _read_copy(s).wait()

            def _cstep(c, carry2):
                coff = c * SK_
                for b in range(BS_):
                    for m in range(NCHUNK_):
                        ts_flat[pl.ds(b * BSTR_ + coff + 16 * m, 16)] = (
                            buf[c, b, pl.ds(c0 + 16 * m, 16)]
                        )
                return carry2

            lax.fori_loop(0, CH_, _cstep, 0)

            s_nxt = s + nworkers

            @pl.when(s_nxt < NSEG_)
            def _():
                _read_copy(s_nxt).start()

            @pl.when(s >= nworkers)
            def _():
                pltpu.make_async_copy(ts_out, out_hbm.at[s - nworkers], wsem).wait()

            off2v = iota * 0 + off2

            def _vstep(v, carry2):
                base = pidx_v[pl.ds(16 * v, 16)] + off2v
                for b in range(BS_):
                    g = plsc.load_gather(ts_flat, [base + (b * BSTR_)])
                    ts_out[b, pl.ds(16 * v, 16)] = g
                return carry2

            lax.fori_loop(0, NVREG_, _vstep, 0)
            pltpu.make_async_copy(ts_out, out_hbm.at[s], wsem).start()

        return carry

    lax.fori_loop(0, SROUNDS_, _seg, 0)

    s_last = wid + nworkers * ((NSEG_ - 1 - wid) // nworkers)
    pltpu.make_async_copy(ts_out, out_hbm.at[s_last], wsem).wait()


def kernel(x, seq_lens):
    xt = jnp.transpose(x, (2, 0, 1))
    sl = seq_lens.astype(jnp.int32)
    didx = jnp.arange(NCH_, dtype=jnp.int32)
    pidx = (didx % CH_) * SK_ + didx // CH_
    mesh = plsc.VectorSubcoreMesh(core_axis_name="c", subcore_axis_name="s")
    sc = functools.partial(
        pl.kernel,
        out_type=[
            jax.ShapeDtypeStruct((NSEG_, BS_, NCH_), x.dtype),
            jax.ShapeDtypeStruct((BS_,), jnp.int32),
        ],
        scratch_types=[
            pltpu.VMEM((CH_, BS_, WIN_), jnp.float32),
            pltpu.VMEM((BS_ * BSTR_,), jnp.float32),
            pltpu.VMEM((BS_, NCH_), jnp.float32),
            pltpu.VMEM((NCH_,), jnp.int32),
            pltpu.VMEM((BS_,), jnp.int32),
            pltpu.SemaphoreType.DMA,
            pltpu.SemaphoreType.DMA,
        ],
        mesh=mesh,
        compiler_params=pltpu.CompilerParams(needs_layout_passes=False),
    )(_sc_body)
    out_t, osl = sc(xt, sl, pidx)
    return jnp.transpose(out_t, (1, 0, 2)), osl.astype(seq_lens.dtype)


# speedup vs baseline: 8.7557x; 1.7725x over previous
import functools

import jax
import jax.numpy as jnp
from jax import lax
from jax.experimental import pallas as pl
from jax.experimental.pallas import tpu as pltpu
from jax.experimental.pallas import tpu_sc as plsc

HZ_ = 100
BS_ = 16
SLEN_ = 30000
CH_ = 12
NSEG_ = SLEN_ // HZ_
NCH_ = CH_ * HZ_
NVREG_ = NCH_ // 16
WIN_ = 256
SK_ = 129
BSTR_ = CH_ * SK_
NCHUNK_ = 8
SROUNDS_ = 10


def _sc_body(x_hbm, sl_hbm, pidx_hbm, out_hbm, osl_hbm,
             buf, ts_flat, ts_out, pidx_v, sl_v, rsem, wsem):
    info = plsc.get_sparse_core_info()
    nworkers = info.num_cores * info.num_subcores
    wid = lax.axis_index("s") * info.num_cores + lax.axis_index("c")
    iota = lax.iota(jnp.int32, 16)

    pltpu.sync_copy(pidx_hbm, pidx_v)

    @pl.when(wid == 0)
    def _():
        pltpu.sync_copy(sl_hbm, sl_v)
        sl_v[...] = lax.div(sl_v[...], iota * 0 + HZ_)
        pltpu.sync_copy(sl_v, osl_hbm)

    def _a0(s):
        return pl.multiple_of(((s * HZ_) // 128) * 128, 128)

    def _read_copy(s):
        return pltpu.make_async_copy(
            x_hbm.at[:, :, pl.ds(_a0(s), WIN_)], buf, rsem
        )

    _read_copy(wid).start()

    def _seg(k, carry):
        s = wid + nworkers * k

        @pl.when(s < NSEG_)
        def _():
            t0 = s * HZ_
            off = t0 - _a0(s)
            c0 = pl.multiple_of((off // 16) * 16, 16)
            off2 = off - c0
            _read_copy(s).wait()

            def _cstep(c, carry2):
                coff = c * SK_
                for b in range(BS_):
                    gs = [
                        buf[c, b, pl.ds(c0 + 16 * m, 16)] for m in range(NCHUNK_)
                    ]
                    for m in range(NCHUNK_):
                        ts_flat[pl.ds(b * BSTR_ + coff + 16 * m, 16)] = gs[m]
                return carry2

            lax.fori_loop(0, CH_, _cstep, 0)

            s_nxt = s + nworkers

            @pl.when(s_nxt < NSEG_)
            def _():
                _read_copy(s_nxt).start()

            @pl.when(s >= nworkers)
            def _():
                pltpu.make_async_copy(ts_out, out_hbm.at[s - nworkers], wsem).wait()

            off2v = iota * 0 + off2

            def _vstep(v, carry2):
                base = pidx_v[pl.ds(16 * v, 16)] + off2v
                gs = [
                    plsc.load_gather(ts_flat, [base + (b * BSTR_)])
                    for b in range(BS_)
                ]
                for b in range(BS_):
                    ts_out[b, pl.ds(16 * v, 16)] = gs[b]
                return carry2

            lax.fori_loop(0, NVREG_, _vstep, 0)
            pltpu.make_async_copy(ts_out, out_hbm.at[s], wsem).start()

        return carry

    lax.fori_loop(0, SROUNDS_, _seg, 0)

    s_last = wid + nworkers * ((NSEG_ - 1 - wid) // nworkers)
    pltpu.make_async_copy(ts_out, out_hbm.at[s_last], wsem).wait()


def kernel(x, seq_lens):
    xt = jnp.transpose(x, (2, 0, 1))
    sl = seq_lens.astype(jnp.int32)
    didx = jnp.arange(NCH_, dtype=jnp.int32)
    pidx = (didx % CH_) * SK_ + didx // CH_
    mesh = plsc.VectorSubcoreMesh(core_axis_name="c", subcore_axis_name="s")
    sc = functools.partial(
        pl.kernel,
        out_type=[
            jax.ShapeDtypeStruct((NSEG_, BS_, NCH_), x.dtype),
            jax.ShapeDtypeStruct((BS_,), jnp.int32),
        ],
        scratch_types=[
            pltpu.VMEM((CH_, BS_, WIN_), jnp.float32),
            pltpu.VMEM((BS_ * BSTR_,), jnp.float32),
            pltpu.VMEM((BS_, NCH_), jnp.float32),
            pltpu.VMEM((NCH_,), jnp.int32),
            pltpu.VMEM((BS_,), jnp.int32),
            pltpu.SemaphoreType.DMA,
            pltpu.SemaphoreType.DMA,
        ],
        mesh=mesh,
        compiler_params=pltpu.CompilerParams(needs_layout_passes=False),
    )(_sc_body)
    out_t, osl = sc(xt, sl, pidx)
    return jnp.transpose(out_t, (1, 0, 2)), osl.astype(seq_lens.dtype)
